# baseline (speedup 1.0000x reference)
import jax
import jax.numpy as jnp
from jax import lax
from jax.experimental import pallas as pl
from jax.experimental.pallas import tpu as pltpu
from jax.experimental.pallas import tpu_sc as plsc

_TOPK = 8
_NB = 8
_B = 128
_E = 32768
_BS = _E // _NB
_NVEC = _BS // 16

_NC = 2
_NS = 16
_NW = _NC * _NS

_RTC = 48
_UNITS = (_B - _RTC) * _NB
_UPW = _UNITS // _NW


def _merge2(a, b):
    c = [jnp.maximum(a[j], b[_TOPK - 1 - j]) for j in range(_TOPK)]
    for d in (4, 2, 1):
        for i in range(_TOPK):
            if i & d:
                continue
            k = i | d
            hi = jnp.maximum(c[i], c[k])
            lo = jnp.minimum(c[i], c[k])
            c[i], c[k] = hi, lo
    return c


_OE8 = [(0, 1), (2, 3), (4, 5), (6, 7),
        (0, 2), (1, 3), (4, 6), (5, 7),
        (1, 2), (5, 6),
        (0, 4), (1, 5), (2, 6), (3, 7),
        (2, 4), (3, 5),
        (1, 2), (3, 4), (5, 6)]


def _sort8_desc(v):
    v = list(v)
    for i, j in _OE8:
        hi = jnp.maximum(v[i], v[j])
        lo = jnp.minimum(v[i], v[j])
        v[i], v[j] = hi, lo
    return v


def _unit_compute(in_ref, out_ref, mat_v, lane, neg, z16):
    nch = 2
    span = _NVEC // nch
    nbatch = span // _TOPK

    def p1(i, r):
        rl = list(r)
        for c in range(nch):
            base = (c * span + i * _TOPK) * 16
            w = _sort8_desc([in_ref[pl.ds(base + s * 16, 16)]
                             for s in range(_TOPK)])
            rl[c * 8:(c + 1) * 8] = _merge2(rl[c * 8:(c + 1) * 8], w)
        return tuple(rl)

    rr = lax.fori_loop(0, nbatch, p1, (neg,) * (8 * nch))
    r = _merge2(list(rr[0:8]), list(rr[8:16]))

    for s in (1, 2, 4, 8):
        partner = lane ^ s
        for j in range(_TOPK):
            mat_v[j] = r[j]
        b = [plsc.load_gather(mat_v,
                              [jnp.full((16,), j, jnp.int32), partner])
             for j in range(_TOPK)]
        r = _merge2(r, b)

    t8 = r[7]
    c_gt = z16
    for j in range(7):
        c_gt = c_gt + (r[j] > t8).astype(jnp.int32)
    need_eq = _TOPK - c_gt

    def p3(i, acc):
        for s in range(8):
            v = in_ref[pl.ds((i * 8 + s) * 16, 16)]
            ge = v >= t8
            out_ref[pl.ds((i * 8 + s) * 16, 16)] = jnp.where(ge, v, 0.0)
            acc = acc + ge.astype(jnp.int32)
        return acc

    acc = lax.fori_loop(0, _NVEC // 8, p3, z16)
    total = jnp.sum(acc)

    @pl.when(total > _TOPK)
    def _fixup():
        def pf(i, run):
            v = in_ref[pl.ds(i * 16, 16)]
            eq = v == t8
            eqc = eq.astype(jnp.int32)
            cum = plsc.cumsum(eqc)
            keep = (v > t8) | (eq & ((cum + run) <= need_eq))
            out_ref[pl.ds(i * 16, 16)] = jnp.where(keep, v, 0.0)
            return run + jnp.sum(eqc)

        lax.fori_loop(0, _NVEC, pf, z16)


def _wta_body(x_hbm, out_hbm, in_v0, in_v1, out_v0, out_v1, mat_v,
              si0, si1, so0, so1):
    wid = lax.axis_index("s") * _NC + lax.axis_index("c")
    lane = lax.iota(jnp.int32, 16)
    neg = jnp.full((16,), -jnp.inf, jnp.float32)
    z16 = jnp.zeros((16,), jnp.int32)

    in_bufs = (in_v0, in_v1)
    out_bufs = (out_v0, out_v1)
    sins = (si0, si1)
    souts = (so0, so1)
    unit0 = wid * _UPW

    def src_at(unit):
        return x_hbm.at[_RTC + unit // _NB, pl.ds((unit % _NB) * _BS, _BS)]

    def dst_at(unit):
        return out_hbm.at[_RTC + unit // _NB, pl.ds((unit % _NB) * _BS, _BS)]

    pltpu.async_copy(src_at(unit0), in_v0, si0)

    def pair_body(h, carry):
        for b in range(2):
            u = 2 * h + b
            unit = unit0 + u

            @pl.when(u + 1 < _UPW)
            def _prefetch():
                pltpu.async_copy(src_at(unit + 1), in_bufs[1 - b],
                                 sins[1 - b])

            pltpu.make_async_copy(src_at(unit), in_bufs[b], sins[b]).wait()

            @pl.when(u >= 2)
            def _drain_out():
                pltpu.make_async_copy(out_bufs[b], dst_at(unit - 2),
                                      souts[b]).wait()

            _unit_compute(in_bufs[b], out_bufs[b], mat_v, lane, neg, z16)
            pltpu.async_copy(out_bufs[b], dst_at(unit), souts[b])
        return carry

    lax.fori_loop(0, _UPW // 2, pair_body, 0)
    pltpu.make_async_copy(out_v0, dst_at(unit0 + _UPW - 2), so0).wait()
    pltpu.make_async_copy(out_v1, dst_at(unit0 + _UPW - 1), so1).wait()


def _tc_body(x_ref, o_ref):
    lane = lax.broadcasted_iota(jnp.int32, (8, 128), 1)
    slices = [x_ref[:, k * 128:(k + 1) * 128] for k in range(32)]
    r = _sort8_desc(slices[0:8])
    for bb in range(1, 4):
        r = _merge2(r, _sort8_desc(slices[bb * 8:(bb + 1) * 8]))
    for s in (1, 2, 4, 8, 16, 32, 64):
        low = (lane & s) == 0
        b = [jnp.where(low, pltpu.roll(r[j], 128 - s, 1),
                       pltpu.roll(r[j], s, 1))
             for j in range(_TOPK)]
        r = _merge2(r, b)
    t8 = r[7]
    c_gt = jnp.zeros((8, 128), jnp.int32)
    for j in range(7):
        c_gt = c_gt + (r[j] > t8).astype(jnp.int32)
    need_eq = _TOPK - c_gt

    acc = jnp.zeros((8, 128), jnp.int32)
    for k in range(32):
        v = slices[k]
        ge = v >= t8
        o_ref[:, k * 128:(k + 1) * 128] = jnp.where(ge, v, 0.0)
        acc = acc + ge.astype(jnp.int32)

    @pl.when(jnp.any(jnp.sum(acc, axis=1) > _TOPK))
    def _fixup():
        runc = jnp.zeros((8, 1), jnp.int32)
        for k in range(32):
            v = x_ref[:, k * 128:(k + 1) * 128]
            eq = v == t8
            p = eq.astype(jnp.int32)
            for sh in (1, 2, 4, 8, 16, 32, 64):
                q = pltpu.roll(p, sh, 1)
                p = p + jnp.where(lane >= sh, q, 0)
            keep = (v > t8) | (eq & ((p + runc) <= need_eq))
            o_ref[:, k * 128:(k + 1) * 128] = jnp.where(keep, v, 0.0)
            runc = runc + jnp.sum(eq.astype(jnp.int32), axis=1, keepdims=True)


def _tc_call(x, interpret=False):
    return pl.pallas_call(
        _tc_body,
        grid=(_RTC // 8, _NB),
        in_specs=[pl.BlockSpec((8, _BS), lambda i, j: (i, j))],
        out_specs=pl.BlockSpec((8, _BS), lambda i, j: (i, j)),
        out_shape=jax.ShapeDtypeStruct((_RTC, _E), jnp.float32),
        compiler_params=pltpu.CompilerParams(
            dimension_semantics=("parallel", "parallel")),
        interpret=interpret,
    )(x)


@jax.jit
def kernel(x):
    mesh = plsc.VectorSubcoreMesh(core_axis_name="c", subcore_axis_name="s")
    f = pl.kernel(
        _wta_body,
        out_type=jax.ShapeDtypeStruct((_B, _E), jnp.float32),
        mesh=mesh,
        scratch_types=[
            pltpu.VMEM((_BS,), jnp.float32),
            pltpu.VMEM((_BS,), jnp.float32),
            pltpu.VMEM((_BS,), jnp.float32),
            pltpu.VMEM((_BS,), jnp.float32),
            pltpu.VMEM((_TOPK, 16), jnp.float32),
            pltpu.SemaphoreType.DMA,
            pltpu.SemaphoreType.DMA,
            pltpu.SemaphoreType.DMA,
            pltpu.SemaphoreType.DMA,
        ],
        compiler_params=pltpu.CompilerParams(needs_layout_passes=False),
    )
    sc_out = f(x)
    tc_out = _tc_call(x)
    return lax.dynamic_update_slice(sc_out, tc_out, (0, 0))


# speedup vs baseline: 1.5157x; 1.5157x over previous
import jax
import jax.numpy as jnp
from jax import lax
from jax.experimental import pallas as pl
from jax.experimental.pallas import tpu as pltpu
from jax.experimental.pallas import tpu_sc as plsc

_TOPK = 8
_NB = 8
_B = 128
_E = 32768
_BS = _E // _NB
_NVEC = _BS // 16

_NC = 2
_NS = 16
_NW = _NC * _NS

_RTC = 48
_UNITS = (_B - _RTC) * _NB
_UPW = _UNITS // _NW


def _merge2(a, b):
    c = [jnp.maximum(a[j], b[_TOPK - 1 - j]) for j in range(_TOPK)]
    for d in (4, 2, 1):
        for i in range(_TOPK):
            if i & d:
                continue
            k = i | d
            hi = jnp.maximum(c[i], c[k])
            lo = jnp.minimum(c[i], c[k])
            c[i], c[k] = hi, lo
    return c


_OE8 = [(0, 1), (2, 3), (4, 5), (6, 7),
        (0, 2), (1, 3), (4, 6), (5, 7),
        (1, 2), (5, 6),
        (0, 4), (1, 5), (2, 6), (3, 7),
        (2, 4), (3, 5),
        (1, 2), (3, 4), (5, 6)]


def _sort8_desc(v):
    v = list(v)
    for i, j in _OE8:
        hi = jnp.maximum(v[i], v[j])
        lo = jnp.minimum(v[i], v[j])
        v[i], v[j] = hi, lo
    return v


def _unit_compute(in_ref, out_ref, mat_v, lane, neg, z16):
    nch = 2
    span = _NVEC // nch
    nbatch = span // _TOPK

    def p1(i, r):
        rl = list(r)
        for c in range(nch):
            base = (c * span + i * _TOPK) * 16
            w = _sort8_desc([in_ref[pl.ds(base + s * 16, 16)]
                             for s in range(_TOPK)])
            rl[c * 8:(c + 1) * 8] = _merge2(rl[c * 8:(c + 1) * 8], w)
        return tuple(rl)

    rr = lax.fori_loop(0, nbatch, p1, (neg,) * (8 * nch))
    r = _merge2(list(rr[0:8]), list(rr[8:16]))

    for s in (1, 2, 4, 8):
        partner = lane ^ s
        for j in range(_TOPK):
            mat_v[j] = r[j]
        b = [plsc.load_gather(mat_v,
                              [jnp.full((16,), j, jnp.int32), partner])
             for j in range(_TOPK)]
        r = _merge2(r, b)

    t8 = r[7]
    c_gt = z16
    for j in range(7):
        c_gt = c_gt + (r[j] > t8).astype(jnp.int32)
    need_eq = _TOPK - c_gt

    def p3(i, acc):
        for s in range(8):
            v = in_ref[pl.ds((i * 8 + s) * 16, 16)]
            ge = v >= t8
            out_ref[pl.ds((i * 8 + s) * 16, 16)] = jnp.where(ge, v, 0.0)
            acc = acc + ge.astype(jnp.int32)
        return acc

    acc = lax.fori_loop(0, _NVEC // 8, p3, z16)
    total = jnp.sum(acc)

    @pl.when(total > _TOPK)
    def _fixup():
        def pf(i, run):
            v = in_ref[pl.ds(i * 16, 16)]
            eq = v == t8
            eqc = eq.astype(jnp.int32)
            cum = plsc.cumsum(eqc)
            keep = (v > t8) | (eq & ((cum + run) <= need_eq))
            out_ref[pl.ds(i * 16, 16)] = jnp.where(keep, v, 0.0)
            return run + jnp.sum(eqc)

        lax.fori_loop(0, _NVEC, pf, z16)


def _wta_body(x_hbm, out_hbm, in_v0, in_v1, out_v0, out_v1, mat_v,
              si0, si1, so0, so1):
    wid = lax.axis_index("s") * _NC + lax.axis_index("c")
    lane = lax.iota(jnp.int32, 16)
    neg = jnp.full((16,), -jnp.inf, jnp.float32)
    z16 = jnp.zeros((16,), jnp.int32)

    in_bufs = (in_v0, in_v1)
    out_bufs = (out_v0, out_v1)
    sins = (si0, si1)
    souts = (so0, so1)
    unit0 = wid * _UPW

    def src_at(unit):
        return x_hbm.at[_RTC + unit // _NB, pl.ds((unit % _NB) * _BS, _BS)]

    def dst_at(unit):
        return out_hbm.at[_RTC + unit // _NB, pl.ds((unit % _NB) * _BS, _BS)]

    pltpu.async_copy(src_at(unit0), in_v0, si0)

    def pair_body(h, carry):
        for b in range(2):
            u = 2 * h + b
            unit = unit0 + u

            @pl.when(u + 1 < _UPW)
            def _prefetch():
                pltpu.async_copy(src_at(unit + 1), in_bufs[1 - b],
                                 sins[1 - b])

            pltpu.make_async_copy(src_at(unit), in_bufs[b], sins[b]).wait()

            @pl.when(u >= 2)
            def _drain_out():
                pltpu.make_async_copy(out_bufs[b], dst_at(unit - 2),
                                      souts[b]).wait()

            _unit_compute(in_bufs[b], out_bufs[b], mat_v, lane, neg, z16)
            pltpu.async_copy(out_bufs[b], dst_at(unit), souts[b])
        return carry

    lax.fori_loop(0, _UPW // 2, pair_body, 0)
    pltpu.make_async_copy(out_v0, dst_at(unit0 + _UPW - 2), so0).wait()
    pltpu.make_async_copy(out_v1, dst_at(unit0 + _UPW - 1), so1).wait()


_GB = 4


def _tc_body(x_ref, o_ref):
    lane = lax.broadcasted_iota(jnp.int32, (8, 128), 1)
    sl = [[x_ref[:, (g * 32 + k) * 128:(g * 32 + k + 1) * 128]
           for k in range(32)] for g in range(_GB)]
    rs = []
    for g in range(_GB):
        r = _sort8_desc(sl[g][0:8])
        for bb in range(1, 4):
            r = _merge2(r, _sort8_desc(sl[g][bb * 8:(bb + 1) * 8]))
        rs.append(r)
    for s in (1, 2, 4, 8, 16, 32, 64):
        low = (lane & s) == 0
        for g in range(_GB):
            b = [jnp.where(low, pltpu.roll(rs[g][j], 128 - s, 1),
                           pltpu.roll(rs[g][j], s, 1))
                 for j in range(_TOPK)]
            rs[g] = _merge2(rs[g], b)

    for g in range(_GB):
        t8 = rs[g][7]
        acc = jnp.zeros((8, 128), jnp.float32)
        for k in range(32):
            v = sl[g][k]
            ge = v >= t8
            o_ref[:, (g * 32 + k) * 128:(g * 32 + k + 1) * 128] = (
                jnp.where(ge, v, 0.0))
            acc = acc + jnp.where(ge, 1.0, 0.0)

        @pl.when(jnp.any(jnp.sum(acc, axis=1) > float(_TOPK)))
        def _fixup(g=g, t8=t8):
            c_gt = jnp.zeros((8, 128), jnp.int32)
            for j in range(7):
                c_gt = c_gt + (rs[g][j] > t8).astype(jnp.int32)
            need_eq = _TOPK - c_gt
            runc = jnp.zeros((8, 1), jnp.int32)
            for k in range(32):
                v = x_ref[:, (g * 32 + k) * 128:(g * 32 + k + 1) * 128]
                eq = v == t8
                p = eq.astype(jnp.int32)
                for sh in (1, 2, 4, 8, 16, 32, 64):
                    q = pltpu.roll(p, sh, 1)
                    p = p + jnp.where(lane >= sh, q, 0)
                keep = (v > t8) | (eq & ((p + runc) <= need_eq))
                o_ref[:, (g * 32 + k) * 128:(g * 32 + k + 1) * 128] = (
                    jnp.where(keep, v, 0.0))
                runc = runc + jnp.sum(eq.astype(jnp.int32), axis=1,
                                      keepdims=True)


def _tc_call(x, interpret=False):
    return pl.pallas_call(
        _tc_body,
        grid=(_RTC // 8, _NB // _GB),
        in_specs=[pl.BlockSpec((8, _GB * _BS), lambda i, j: (i, j))],
        out_specs=pl.BlockSpec((8, _GB * _BS), lambda i, j: (i, j)),
        out_shape=jax.ShapeDtypeStruct((_RTC, _E), jnp.float32),
        compiler_params=pltpu.CompilerParams(
            dimension_semantics=("parallel", "parallel")),
        interpret=interpret,
    )(x)


@jax.jit
def kernel(x):
    mesh = plsc.VectorSubcoreMesh(core_axis_name="c", subcore_axis_name="s")
    f = pl.kernel(
        _wta_body,
        out_type=jax.ShapeDtypeStruct((_B, _E), jnp.float32),
        mesh=mesh,
        scratch_types=[
            pltpu.VMEM((_BS,), jnp.float32),
            pltpu.VMEM((_BS,), jnp.float32),
            pltpu.VMEM((_BS,), jnp.float32),
            pltpu.VMEM((_BS,), jnp.float32),
            pltpu.VMEM((_TOPK, 16), jnp.float32),
            pltpu.SemaphoreType.DMA,
            pltpu.SemaphoreType.DMA,
            pltpu.SemaphoreType.DMA,
            pltpu.SemaphoreType.DMA,
        ],
        compiler_params=pltpu.CompilerParams(needs_layout_passes=False),
    )
    sc_out = f(x)
    tc_out = _tc_call(x)
    return lax.dynamic_update_slice(sc_out, tc_out, (0, 0))
